# baseline (speedup 1.0000x reference)
import functools

import jax
import jax.numpy as jnp
from jax import lax
from jax.experimental import pallas as pl
from jax.experimental.pallas import tpu as pltpu
from jax.experimental.pallas import tpu_sc as plsc

_INFO = plsc.get_sparse_core_info()
_NC = _INFO.num_cores
_NS = _INFO.num_subcores
_NW = _NC * _NS


def _make_sc_copy_concat(B, D, R, nbuf, lag):
    assert B % _NW == 0
    rpw = B // _NW
    assert rpw % R == 0
    n_chunks = rpw // R
    lag = min(lag, nbuf - 1)
    mesh = plsc.VectorSubcoreMesh(core_axis_name="c", subcore_axis_name="s")

    @functools.partial(
        pl.kernel,
        mesh=mesh,
        out_type=jax.ShapeDtypeStruct((B, 2 * D), jnp.float32),
        compiler_params=pltpu.CompilerParams(use_tc_tiling_on_sc=True),
        scratch_types=[
            pltpu.VMEM((R,), jnp.int32),
            pltpu.VMEM((nbuf, R, 2 * D), jnp.float32),
            pltpu.SemaphoreType.DMA,
            pltpu.SemaphoreType.DMA((nbuf,)),
            pltpu.SemaphoreType.DMA((nbuf,)),
        ],
    )
    def k(x_hbm, pos_hbm, idx_hbm, out_hbm, idx_v, bufs, pf_sem, in_sems,
          out_sems):
        wid = lax.axis_index("s") * _NC + lax.axis_index("c")
        base = wid * rpw

        def in_copy(c):
            b = c % nbuf
            return pltpu.make_async_copy(
                x_hbm.at[pl.ds(base + c * R, R), :],
                bufs.at[b, :, pl.ds(0, D)],
                in_sems.at[b],
            )

        def out_copy(c):
            b = c % nbuf
            return pltpu.make_async_copy(
                bufs.at[b],
                out_hbm.at[pl.ds(base + c * R, R), :],
                out_sems.at[b],
            )

        pltpu.sync_copy(idx_hbm, idx_v)
        pf = [
            pltpu.make_async_copy(
                pos_hbm.at[idx_v], bufs.at[b, :, pl.ds(D, D)], pf_sem)
            for b in range(nbuf)
        ]
        for c in pf:
            c.start()
        for c in range(min(nbuf, n_chunks)):
            in_copy(c).start()
        for c in pf:
            c.wait()
        started_in = min(nbuf, n_chunks)
        waited_out = 0
        for c in range(n_chunks):
            in_copy(c).wait()
            out_copy(c).start()
            p = c - lag
            nxt = p + nbuf
            if p >= 0 and nxt == started_in and nxt < n_chunks:
                out_copy(p).wait()
                waited_out = p + 1
                in_copy(nxt).start()
                started_in = nxt + 1
        for c in range(waited_out, n_chunks):
            out_copy(c).wait()

    return k


def kernel(x, pos_table, index):
    B, _, D = x.shape
    x2 = x.reshape(B, D)
    pos2 = pos_table.reshape(pos_table.shape[1], D)
    R = 32
    idx = jnp.broadcast_to(jnp.asarray(index, jnp.int32).reshape(1), (R,))
    out = _make_sc_copy_concat(B, D, R, nbuf=7, lag=4)(x2, pos2, idx)
    return out.reshape(B, 1, 2 * D)


# speedup vs baseline: 4.4996x; 4.4996x over previous
import jax
import jax.numpy as jnp
from jax.experimental import pallas as pl
from jax.experimental.pallas import tpu as pltpu


def _body(idx_ref, x_ref, pos_ref, out_ref):
    del idx_ref
    out_ref[:, 0:256] = x_ref[...]
    out_ref[:, 256:512] = jnp.broadcast_to(
        pos_ref[0], out_ref[:, 256:512].shape)


def kernel(x, pos_table, index):
    B, _, D = x.shape
    x2 = x.reshape(B, D)
    pos3 = pos_table.reshape(pos_table.shape[1], 1, D)
    bm = 1024
    grid = B // bm
    idx = jnp.asarray(index, jnp.int32).reshape(1)
    out = pl.pallas_call(
        _body,
        grid_spec=pltpu.PrefetchScalarGridSpec(
            num_scalar_prefetch=1,
            grid=(grid,),
            in_specs=[
                pl.BlockSpec((bm, D), lambda i, s: (i, 0)),
                pl.BlockSpec((1, 1, D), lambda i, s: (s[0], 0, 0)),
            ],
            out_specs=pl.BlockSpec((bm, 2 * D), lambda i, s: (i, 0)),
        ),
        out_shape=jax.ShapeDtypeStruct((B, 2 * D), jnp.float32),
        compiler_params=pltpu.CompilerParams(
            dimension_semantics=("arbitrary",),
        ),
    )(idx, x2, pos3)
    return out.reshape(B, 1, 2 * D)
